# baseline (speedup 1.0000x reference)
import functools

import jax
import jax.numpy as jnp
from jax import lax
from jax.experimental import pallas as pl
from jax.experimental.pallas import tpu as pltpu
from jax.experimental.pallas import tpu_sc as plsc

N = 10000
E = 320000
D = 128
H1 = 256

NUM_TILES = 16
N_PAD = 10112
ROWS_PER_TILE = N_PAD // NUM_TILES
CHUNK = 128
NUM_CHUNKS = E // CHUNK
CHUNKS_PER_TILE = NUM_CHUNKS // NUM_TILES
NBUF = 3


def _spmm_body(m1_hbm, m2_hbm, src1_hbm, dst1_hbm, src2_hbm, dst2_hbm,
               zeros_hbm, out1_hbm, out2_hbm,
               acc_sh, rows0, rows1, rows2, sidx0, sidx1, sidx2,
               didx0, didx1, didx2,
               gsem0, gsem1, gsem2, ssem0, ssem1, ssem2,
               dsem0, dsem1, dsem2, xsem0, xsem1, xsem2, zsem):
  c = lax.axis_index("c")
  s = lax.axis_index("s")
  row_base = s * ROWS_PER_TILE
  rows = (rows0, rows1, rows2)
  sidx = (sidx0, sidx1, sidx2)
  didx = (didx0, didx1, didx2)
  gsem = (gsem0, gsem1, gsem2)
  ssem = (ssem0, ssem1, ssem2)
  dsem = (dsem0, dsem1, dsem2)
  xsem = (xsem0, xsem1, xsem2)
  NC = CHUNKS_PER_TILE

  pltpu.async_copy(zeros_hbm, acc_sh.at[pl.ds(row_base, ROWS_PER_TILE)], zsem)

  def do_edges(m_hbm, src_hbm, dst_hbm):
    base_e = s * CHUNKS_PER_TILE * CHUNK

    def idx_wait(buf, sem):
      pltpu.make_async_copy(src_hbm.at[pl.ds(0, CHUNK)], buf, sem).wait()

    def row_wait(buf, sem):
      pltpu.make_async_copy(m_hbm.at[sidx0], buf, sem).wait()

    def scat_wait(k):
      pltpu.make_async_copy(rows[k], acc_sh.at[didx[k]], xsem[k]).wait()

    def load_sidx(i, k):
      pltpu.async_copy(src_hbm.at[pl.ds(base_e + i * CHUNK, CHUNK)],
                       sidx[k], ssem[k])

    def load_didx(i, k):
      pltpu.async_copy(dst_hbm.at[pl.ds(base_e + i * CHUNK, CHUNK)],
                       didx[k], dsem[k])

    def issue_gather(k):
      pltpu.async_copy(m_hbm.at[sidx[k]], rows[k], gsem[k])

    for k in range(NBUF):
      load_sidx(k, k)
      if k < NBUF - 1:
        load_didx(k, k)
    for k in range(NBUF - 1):
      idx_wait(sidx[k], ssem[k])
      issue_gather(k)

    pltpu.make_async_copy(zeros_hbm, acc_sh.at[pl.ds(0, ROWS_PER_TILE)],
                          zsem).wait()
    plsc.subcore_barrier()

    def step(i, k):
      o = (k + 2) % NBUF
      row_wait(rows[k], gsem[k])

      @pl.when(i >= 1)
      def _():
        scat_wait(o)

      @pl.when(i + 2 < NC)
      def _():
        idx_wait(sidx[o], ssem[o])
        issue_gather(o)

      @pl.when(i + 3 < NC)
      def _():
        load_sidx(i + 3, k)

      @pl.when(i + 2 < NC)
      def _():
        load_didx(i + 2, o)

      idx_wait(didx[k], dsem[k])
      pltpu.async_copy(rows[k], acc_sh.at[didx[k]], xsem[k], add=True)

    def body3(j, carry):
      i = j * NBUF
      for k in range(NBUF):
        step(i + k, k)
      return carry

    lax.fori_loop(0, NC // NBUF, body3, 0)
    scat_wait((NC - 1) % NBUF)

    @pl.when(s < NUM_CHUNKS - NC * NUM_TILES)
    def _():
      off = (NC * NUM_TILES + s) * CHUNK
      pltpu.sync_copy(src_hbm.at[pl.ds(off, CHUNK)], sidx[0])
      pltpu.sync_copy(dst_hbm.at[pl.ds(off, CHUNK)], didx[0])
      pltpu.async_copy(m_hbm.at[sidx[0]], rows[0], gsem[0]).wait()
      pltpu.sync_copy(rows[0], acc_sh.at[didx[0]], add=True)

  @pl.when(c == 0)
  def _():
    do_edges(m1_hbm, src1_hbm, dst1_hbm)

  @pl.when(c == 1)
  def _():
    do_edges(m2_hbm, src2_hbm, dst2_hbm)

  plsc.subcore_barrier()

  @pl.when(c == 0)
  def _():
    pltpu.sync_copy(acc_sh.at[pl.ds(row_base, ROWS_PER_TILE)],
                    out1_hbm.at[pl.ds(row_base, ROWS_PER_TILE)])

  @pl.when(c == 1)
  def _():
    pltpu.sync_copy(acc_sh.at[pl.ds(row_base, ROWS_PER_TILE)],
                    out2_hbm.at[pl.ds(row_base, ROWS_PER_TILE)])


_spmm = pl.kernel(
    _spmm_body,
    out_type=(jax.ShapeDtypeStruct((N_PAD, D), jnp.float32),
              jax.ShapeDtypeStruct((N_PAD, D), jnp.float32)),
    mesh=plsc.VectorSubcoreMesh(core_axis_name="c", subcore_axis_name="s"),
    scratch_types=(
        [pltpu.VMEM_SHARED((N_PAD, D), jnp.float32)]
        + [pltpu.VMEM((CHUNK, D), jnp.float32)] * NBUF
        + [pltpu.VMEM((CHUNK,), jnp.int32)] * (2 * NBUF)
        + [pltpu.SemaphoreType.DMA] * (4 * NBUF + 1)
    ),
)


ROW_BLK = 2000


def _dense1_body(s1_ref, s2_ref, w11_ref, w12_ref, w21_ref, w22_ref,
                 m1_ref, m2_ref):
  h = jnp.maximum(
      jnp.dot(s1_ref[...], w11_ref[...], preferred_element_type=jnp.float32)
      + jnp.dot(s2_ref[...], w12_ref[...], preferred_element_type=jnp.float32),
      0.0)
  m1_ref[...] = jnp.dot(h, w21_ref[...], preferred_element_type=jnp.float32)
  m2_ref[...] = jnp.dot(h, w22_ref[...], preferred_element_type=jnp.float32)


def _dense1(s1, s2, w11, w12, w21, w22):
  grid = N // ROW_BLK
  return pl.pallas_call(
      _dense1_body,
      grid=(grid,),
      in_specs=[
          pl.BlockSpec((ROW_BLK, D), lambda i: (i, 0)),
          pl.BlockSpec((ROW_BLK, D), lambda i: (i, 0)),
          pl.BlockSpec((D, H1), lambda i: (0, 0)),
          pl.BlockSpec((D, H1), lambda i: (0, 0)),
          pl.BlockSpec((H1, D), lambda i: (0, 0)),
          pl.BlockSpec((H1, D), lambda i: (0, 0)),
      ],
      out_specs=[
          pl.BlockSpec((ROW_BLK, D), lambda i: (i, 0)),
          pl.BlockSpec((ROW_BLK, D), lambda i: (i, 0)),
      ],
      out_shape=[
          jax.ShapeDtypeStruct((N, D), jnp.float32),
          jax.ShapeDtypeStruct((N, D), jnp.float32),
      ],
  )(s1, s2, w11, w12, w21, w22)


def _dense2_body(t1_ref, t2_ref, out_ref):
  @pl.when(pl.program_id(0) == 0)
  def _():
    out_ref[...] = jnp.zeros_like(out_ref)

  h2 = jnp.maximum(t1_ref[...] + t2_ref[...], 0.0)
  out_ref[...] += jnp.sum(h2, axis=0, keepdims=True)

  @pl.when(pl.program_id(0) == pl.num_programs(0) - 1)
  def _():
    out_ref[...] *= (1.0 / N)


def _dense2(t1, t2):
  grid = N // ROW_BLK
  return pl.pallas_call(
      _dense2_body,
      grid=(grid,),
      in_specs=[
          pl.BlockSpec((ROW_BLK, D), lambda i: (i, 0)),
          pl.BlockSpec((ROW_BLK, D), lambda i: (i, 0)),
      ],
      out_specs=pl.BlockSpec((1, D), lambda i: (0, 0)),
      out_shape=jax.ShapeDtypeStruct((1, D), jnp.float32),
  )(t1, t2)


def kernel(x, edge_index_1, edge_index_2, W1_r1, W1_r2, W2_r1, W2_r2):
  src1, dst1 = edge_index_1[0], edge_index_1[1]
  src2, dst2 = edge_index_2[0], edge_index_2[1]
  zeros = jnp.zeros((ROWS_PER_TILE, D), jnp.float32)

  s1, s2 = _spmm(x, x, src1, dst1, src2, dst2, zeros)
  m1, m2 = _dense1(s1, s2, W1_r1, W1_r2, W2_r1, W2_r2)
  t1, t2 = _spmm(m1, m2, src1, dst1, src2, dst2, zeros)
  return _dense2(t1, t2)


# speedup vs baseline: 1.0084x; 1.0084x over previous
import functools

import jax
import jax.numpy as jnp
from jax import lax
from jax.experimental import pallas as pl
from jax.experimental.pallas import tpu as pltpu
from jax.experimental.pallas import tpu_sc as plsc

N = 10000
E = 320000
D = 128
H1 = 256

NUM_TILES = 16
N_PAD = 10112
ROWS_PER_TILE = N_PAD // NUM_TILES
CHUNK = 128
NUM_CHUNKS = E // CHUNK
CHUNKS_PER_TILE = NUM_CHUNKS // NUM_TILES
NBUF = 3


def _spmm_body(m1_hbm, m2_hbm, src1_hbm, dst1_hbm, src2_hbm, dst2_hbm,
               zeros_hbm, out1_hbm, out2_hbm,
               acc_sh, rows0, rows1, rows2, sidx0, sidx1, sidx2,
               didx0, didx1, didx2,
               gsem0, gsem1, gsem2, ssem0, ssem1, ssem2,
               dsem0, dsem1, dsem2, xsem0, xsem1, xsem2, zsem):
  c = lax.axis_index("c")
  s = lax.axis_index("s")
  row_base = s * ROWS_PER_TILE
  rows = (rows0, rows1, rows2)
  sidx = (sidx0, sidx1, sidx2)
  didx = (didx0, didx1, didx2)
  gsem = (gsem0, gsem1, gsem2)
  ssem = (ssem0, ssem1, ssem2)
  dsem = (dsem0, dsem1, dsem2)
  xsem = (xsem0, xsem1, xsem2)
  NC = CHUNKS_PER_TILE

  pltpu.async_copy(zeros_hbm, acc_sh.at[pl.ds(row_base, ROWS_PER_TILE)], zsem)

  def do_edges(m_hbm, src_hbm, dst_hbm):
    base_e = s * CHUNKS_PER_TILE * CHUNK

    def idx_wait(buf, sem):
      pltpu.make_async_copy(src_hbm.at[pl.ds(0, CHUNK)], buf, sem).wait()

    def row_wait(buf, sem):
      pltpu.make_async_copy(m_hbm.at[sidx0], buf, sem).wait()

    def scat_wait(k):
      pltpu.make_async_copy(rows[k], acc_sh.at[didx[k]], xsem[k]).wait()

    def load_sidx(i, k):
      pltpu.async_copy(src_hbm.at[pl.ds(base_e + i * CHUNK, CHUNK)],
                       sidx[k], ssem[k])

    def load_didx(i, k):
      pltpu.async_copy(dst_hbm.at[pl.ds(base_e + i * CHUNK, CHUNK)],
                       didx[k], dsem[k])

    def issue_gather(k):
      pltpu.async_copy(m_hbm.at[sidx[k]], rows[k], gsem[k])

    for k in range(NBUF):
      load_sidx(k, k)
      if k < NBUF - 1:
        load_didx(k, k)
    for k in range(NBUF - 1):
      idx_wait(sidx[k], ssem[k])
      issue_gather(k)

    pltpu.make_async_copy(zeros_hbm, acc_sh.at[pl.ds(0, ROWS_PER_TILE)],
                          zsem).wait()
    plsc.subcore_barrier()

    def step(i, k):
      o = (k + 2) % NBUF
      row_wait(rows[k], gsem[k])

      @pl.when(i >= 1)
      def _():
        scat_wait(o)

      @pl.when(i + 2 < NC)
      def _():
        idx_wait(sidx[o], ssem[o])
        issue_gather(o)

      @pl.when(i + 3 < NC)
      def _():
        load_sidx(i + 3, k)

      @pl.when(i + 2 < NC)
      def _():
        load_didx(i + 2, o)

      idx_wait(didx[k], dsem[k])
      pltpu.async_copy(rows[k], acc_sh.at[didx[k]], xsem[k], add=True)

      tail_off = (NC * NUM_TILES + s) * CHUNK

      @pl.when((i == NC - 2) & (s < NUM_CHUNKS - NC * NUM_TILES))
      def _():
        pltpu.async_copy(src_hbm.at[pl.ds(tail_off, CHUNK)], sidx[0], ssem[0])
        pltpu.async_copy(dst_hbm.at[pl.ds(tail_off, CHUNK)], didx[0], dsem[0])

      @pl.when((i == NC - 1) & (s < NUM_CHUNKS - NC * NUM_TILES))
      def _():
        idx_wait(sidx[0], ssem[0])
        issue_gather(0)

    def body3(j, carry):
      i = j * NBUF
      for k in range(NBUF):
        step(i + k, k)
      return carry

    lax.fori_loop(0, NC // NBUF, body3, 0)
    scat_wait((NC - 1) % NBUF)

    @pl.when(s < NUM_CHUNKS - NC * NUM_TILES)
    def _():
      row_wait(rows[0], gsem[0])
      idx_wait(didx[0], dsem[0])
      pltpu.sync_copy(rows[0], acc_sh.at[didx[0]], add=True)

  @pl.when(c == 0)
  def _():
    do_edges(m1_hbm, src1_hbm, dst1_hbm)

  @pl.when(c == 1)
  def _():
    do_edges(m2_hbm, src2_hbm, dst2_hbm)

  plsc.subcore_barrier()

  @pl.when(c == 0)
  def _():
    pltpu.sync_copy(acc_sh.at[pl.ds(row_base, ROWS_PER_TILE)],
                    out1_hbm.at[pl.ds(row_base, ROWS_PER_TILE)])

  @pl.when(c == 1)
  def _():
    pltpu.sync_copy(acc_sh.at[pl.ds(row_base, ROWS_PER_TILE)],
                    out2_hbm.at[pl.ds(row_base, ROWS_PER_TILE)])


_spmm = pl.kernel(
    _spmm_body,
    out_type=(jax.ShapeDtypeStruct((N_PAD, D), jnp.float32),
              jax.ShapeDtypeStruct((N_PAD, D), jnp.float32)),
    mesh=plsc.VectorSubcoreMesh(core_axis_name="c", subcore_axis_name="s"),
    scratch_types=(
        [pltpu.VMEM_SHARED((N_PAD, D), jnp.float32)]
        + [pltpu.VMEM((CHUNK, D), jnp.float32)] * NBUF
        + [pltpu.VMEM((CHUNK,), jnp.int32)] * (2 * NBUF)
        + [pltpu.SemaphoreType.DMA] * (4 * NBUF + 1)
    ),
)


ROW_BLK = 2000


def _dense1_body(s1_ref, s2_ref, w11_ref, w12_ref, w21_ref, w22_ref,
                 m1_ref, m2_ref):
  h = jnp.maximum(
      jnp.dot(s1_ref[...], w11_ref[...], preferred_element_type=jnp.float32)
      + jnp.dot(s2_ref[...], w12_ref[...], preferred_element_type=jnp.float32),
      0.0)
  m1_ref[...] = jnp.dot(h, w21_ref[...], preferred_element_type=jnp.float32)
  m2_ref[...] = jnp.dot(h, w22_ref[...], preferred_element_type=jnp.float32)


def _dense1(s1, s2, w11, w12, w21, w22):
  grid = N // ROW_BLK
  return pl.pallas_call(
      _dense1_body,
      grid=(grid,),
      in_specs=[
          pl.BlockSpec((ROW_BLK, D), lambda i: (i, 0)),
          pl.BlockSpec((ROW_BLK, D), lambda i: (i, 0)),
          pl.BlockSpec((D, H1), lambda i: (0, 0)),
          pl.BlockSpec((D, H1), lambda i: (0, 0)),
          pl.BlockSpec((H1, D), lambda i: (0, 0)),
          pl.BlockSpec((H1, D), lambda i: (0, 0)),
      ],
      out_specs=[
          pl.BlockSpec((ROW_BLK, D), lambda i: (i, 0)),
          pl.BlockSpec((ROW_BLK, D), lambda i: (i, 0)),
      ],
      out_shape=[
          jax.ShapeDtypeStruct((N, D), jnp.float32),
          jax.ShapeDtypeStruct((N, D), jnp.float32),
      ],
  )(s1, s2, w11, w12, w21, w22)


def _dense2_body(t1_ref, t2_ref, out_ref):
  @pl.when(pl.program_id(0) == 0)
  def _():
    out_ref[...] = jnp.zeros_like(out_ref)

  h2 = jnp.maximum(t1_ref[...] + t2_ref[...], 0.0)
  out_ref[...] += jnp.sum(h2, axis=0, keepdims=True)

  @pl.when(pl.program_id(0) == pl.num_programs(0) - 1)
  def _():
    out_ref[...] *= (1.0 / N)


def _dense2(t1, t2):
  grid = N // ROW_BLK
  return pl.pallas_call(
      _dense2_body,
      grid=(grid,),
      in_specs=[
          pl.BlockSpec((ROW_BLK, D), lambda i: (i, 0)),
          pl.BlockSpec((ROW_BLK, D), lambda i: (i, 0)),
      ],
      out_specs=pl.BlockSpec((1, D), lambda i: (0, 0)),
      out_shape=jax.ShapeDtypeStruct((1, D), jnp.float32),
  )(t1, t2)


def kernel(x, edge_index_1, edge_index_2, W1_r1, W1_r2, W2_r1, W2_r2):
  src1, dst1 = edge_index_1[0], edge_index_1[1]
  src2, dst2 = edge_index_2[0], edge_index_2[1]
  zeros = jnp.zeros((ROWS_PER_TILE, D), jnp.float32)

  s1, s2 = _spmm(x, x, src1, dst1, src2, dst2, zeros)
  m1, m2 = _dense1(s1, s2, W1_r1, W1_r2, W2_r1, W2_r2)
  t1, t2 = _spmm(m1, m2, src1, dst1, src2, dst2, zeros)
  return _dense2(t1, t2)
